# baseline (speedup 1.0000x reference)
import jax
import jax.numpy as jnp
from jax.experimental import pallas as pl
from jax.experimental.pallas import tpu as pltpu

_K = 68
_HW = 64 * 64


def _rsum(x):
    h, w, l = x.shape
    ones = jnp.ones((1, h * w), jnp.float32)
    return jnp.dot(ones, x.reshape(h * w, l),
                   preferred_element_type=jnp.float32)[0]


def _stage1_kernel(o11_ref, o12_ref, o21_ref, o22_ref, t1_ref, t2_ref,
                   r13_ref, r24_ref):
    t1 = t1_ref[0]
    t1d = jnp.concatenate([t1, t1], axis=-1)
    t2 = t2_ref[0]

    d1 = o11_ref[0] - t1
    r1 = _rsum(d1 * d1)
    d3 = o21_ref[0] - t1
    r3 = _rsum(d3 * d3)
    p2 = o12_ref[0] * t1d - t2
    r2f = _rsum(p2 * p2)
    p4 = (o22_ref[0] - t2) * t1d
    r4f = _rsum(p4 * p4)

    r13_ref[0] = jnp.stack([r1, r3], axis=0)
    r24_ref[0] = jnp.stack([r2f, r4f], axis=0)


def _stage2_kernel(r13_ref, r24_ref, w_ref, o_ref):
    B = r13_ref.shape[0]
    K = r13_ref.shape[2]
    r1 = r13_ref[:, 0, :]
    r3 = r13_ref[:, 1, :]
    r2 = r24_ref[:, 0, :K] + r24_ref[:, 0, K:]
    r4 = r24_ref[:, 1, :K] + r24_ref[:, 1, K:]

    v = r3 * (0.5 / B)

    vi = v[:, :, None]
    vj = v[:, None, :]
    jj = jax.lax.broadcasted_iota(jnp.int32, (B, K, K), 2)
    kk = jax.lax.broadcasted_iota(jnp.int32, (B, K, K), 1)
    beats = (vj > vi) | ((vj == vi) & (jj < kk))
    rank = jnp.sum(beats.astype(jnp.int32), axis=2)
    mask = (rank < (K // 2)).astype(jnp.float32)

    inv_n = 1.0 / (B * K * _HW)
    loss1_1 = jnp.sum(r1) * inv_n
    loss1_2 = jnp.sum(r2) * inv_n
    loss2_1_m = jnp.sum(v * mask) / (B * K)
    loss2_2 = jnp.sum(r4 * mask) * inv_n

    w = w_ref[...]
    loss = ((loss1_1 + loss2_1_m) * w[0, 0]
            + (loss1_2 + loss2_2 * 5.0) * w[0, 1])
    o_ref[...] = jnp.reshape(loss, (1, 1))


def kernel(out1_1, out1_2, out2_1, out2_2, targets1, targets2, weights):
    B, K, H, W = out1_1.shape

    def t(x):
        return jnp.transpose(x, (0, 2, 3, 1))

    spec1 = pl.BlockSpec((1, H, W, K), lambda i: (i, 0, 0, 0))
    spec2 = pl.BlockSpec((1, H, W, 2 * K), lambda i: (i, 0, 0, 0))

    r13, r24 = pl.pallas_call(
        _stage1_kernel,
        grid=(B,),
        in_specs=[spec1, spec2, spec1, spec2, spec1, spec2],
        out_specs=[
            pl.BlockSpec((1, 2, K), lambda i: (i, 0, 0)),
            pl.BlockSpec((1, 2, 2 * K), lambda i: (i, 0, 0)),
        ],
        out_shape=[
            jax.ShapeDtypeStruct((B, 2, K), jnp.float32),
            jax.ShapeDtypeStruct((B, 2, 2 * K), jnp.float32),
        ],
        compiler_params=pltpu.CompilerParams(
            dimension_semantics=("arbitrary",),
        ),
    )(t(out1_1), t(out1_2), t(out2_1), t(out2_2), t(targets1), t(targets2))

    loss = pl.pallas_call(
        _stage2_kernel,
        in_specs=[
            pl.BlockSpec((B, 2, K), lambda: (0, 0, 0)),
            pl.BlockSpec((B, 2, 2 * K), lambda: (0, 0, 0)),
            pl.BlockSpec((1, 2), lambda: (0, 0)),
        ],
        out_specs=pl.BlockSpec((1, 1), lambda: (0, 0)),
        out_shape=jax.ShapeDtypeStruct((1, 1), jnp.float32),
    )(r13, r24, weights.reshape(1, 2))

    return loss[0, 0]


# speedup vs baseline: 1.0056x; 1.0056x over previous
import jax
import jax.numpy as jnp
from jax.experimental import pallas as pl
from jax.experimental.pallas import tpu as pltpu

_K = 68
_HW = 64 * 64


def _rsum(x):
    return jnp.sum(x, axis=(0, 1))


def _stage1_kernel(o11_ref, o12_ref, o21_ref, o22_ref, t1_ref, t2_ref,
                   r13_ref, r24_ref):
    t1 = t1_ref[0]
    t1d = jnp.concatenate([t1, t1], axis=-1)
    t2 = t2_ref[0]

    d1 = o11_ref[0] - t1
    r1 = _rsum(d1 * d1)
    d3 = o21_ref[0] - t1
    r3 = _rsum(d3 * d3)
    p2 = o12_ref[0] * t1d - t2
    r2f = _rsum(p2 * p2)
    p4 = (o22_ref[0] - t2) * t1d
    r4f = _rsum(p4 * p4)

    r13_ref[0] = jnp.stack([r1, r3], axis=0)
    r24_ref[0] = jnp.stack([r2f, r4f], axis=0)


def _stage2_kernel(r13_ref, r24_ref, w_ref, o_ref):
    B = r13_ref.shape[0]
    K = r13_ref.shape[2]
    r1 = r13_ref[:, 0, :]
    r3 = r13_ref[:, 1, :]
    r2 = r24_ref[:, 0, :K] + r24_ref[:, 0, K:]
    r4 = r24_ref[:, 1, :K] + r24_ref[:, 1, K:]

    v = r3 * (0.5 / B)

    vi = v[:, :, None]
    vj = v[:, None, :]
    jj = jax.lax.broadcasted_iota(jnp.int32, (B, K, K), 2)
    kk = jax.lax.broadcasted_iota(jnp.int32, (B, K, K), 1)
    beats = (vj > vi) | ((vj == vi) & (jj < kk))
    rank = jnp.sum(beats.astype(jnp.int32), axis=2)
    mask = (rank < (K // 2)).astype(jnp.float32)

    inv_n = 1.0 / (B * K * _HW)
    loss1_1 = jnp.sum(r1) * inv_n
    loss1_2 = jnp.sum(r2) * inv_n
    loss2_1_m = jnp.sum(v * mask) / (B * K)
    loss2_2 = jnp.sum(r4 * mask) * inv_n

    w = w_ref[...]
    loss = ((loss1_1 + loss2_1_m) * w[0, 0]
            + (loss1_2 + loss2_2 * 5.0) * w[0, 1])
    o_ref[...] = jnp.reshape(loss, (1, 1))


def kernel(out1_1, out1_2, out2_1, out2_2, targets1, targets2, weights):
    B, K, H, W = out1_1.shape

    def t(x):
        return jnp.transpose(x, (0, 2, 3, 1))

    spec1 = pl.BlockSpec((1, H, W, K), lambda i: (i, 0, 0, 0))
    spec2 = pl.BlockSpec((1, H, W, 2 * K), lambda i: (i, 0, 0, 0))

    r13, r24 = pl.pallas_call(
        _stage1_kernel,
        grid=(B,),
        in_specs=[spec1, spec2, spec1, spec2, spec1, spec2],
        out_specs=[
            pl.BlockSpec((1, 2, K), lambda i: (i, 0, 0)),
            pl.BlockSpec((1, 2, 2 * K), lambda i: (i, 0, 0)),
        ],
        out_shape=[
            jax.ShapeDtypeStruct((B, 2, K), jnp.float32),
            jax.ShapeDtypeStruct((B, 2, 2 * K), jnp.float32),
        ],
        compiler_params=pltpu.CompilerParams(
            dimension_semantics=("parallel",),
        ),
    )(t(out1_1), t(out1_2), t(out2_1), t(out2_2), t(targets1), t(targets2))

    loss = pl.pallas_call(
        _stage2_kernel,
        in_specs=[
            pl.BlockSpec((B, 2, K), lambda: (0, 0, 0)),
            pl.BlockSpec((B, 2, 2 * K), lambda: (0, 0, 0)),
            pl.BlockSpec((1, 2), lambda: (0, 0)),
        ],
        out_specs=pl.BlockSpec((1, 1), lambda: (0, 0)),
        out_shape=jax.ShapeDtypeStruct((1, 1), jnp.float32),
    )(r13, r24, weights.reshape(1, 2))

    return loss[0, 0]
